# baseline (speedup 1.0000x reference)
import jax
import jax.numpy as jnp
from jax import lax
from jax.experimental import pallas as pl
from jax.experimental.pallas import tpu as pltpu

_K = 1024
_D = 256
_N = 8192
_BM = 1024
_NB = _N // _BM
_E_WEIGHT = 0.25
_MANAGE_WEIGHT = 0.1


def _vq_body(x_ref, emb_ref, out_ref, stats_ref, sump_ref, acc_ref):
    i = pl.program_id(0)

    @pl.when(i == 0)
    def _init():
        sump_ref[...] = jnp.zeros_like(sump_ref)
        acc_ref[0] = 0.0
        acc_ref[1] = 0.0

    xt = x_ref[0]
    emb = emb_ref[...]
    norm = jnp.sqrt(jnp.sum(xt * xt, axis=0, keepdims=True))
    xn = xt / jnp.maximum(norm, 1e-12)
    s = jnp.sum(xn * xn, axis=0, keepdims=True)
    e2 = jnp.sum(emb * emb, axis=1, keepdims=True)
    xm2 = xn * (-2.0)
    dots2 = lax.dot_general(emb, xm2, (((1,), (0,)), ((), ())))
    d = (e2 + s) + dots2

    ex = jnp.exp(d)
    se = jnp.sum(ex, axis=0, keepdims=True)
    exd = jnp.sum(ex * d, axis=0, keepdims=True)
    r = 1.0 / se
    ent_row = jnp.log(se) - exd * r

    mind = jnp.min(d, axis=0, keepdims=True)
    kio = lax.broadcasted_iota(jnp.int32, d.shape, 0)
    first = jnp.min(jnp.where(d == mind, kio, _K), axis=0, keepdims=True)
    oh = (kio == first).astype(jnp.float32)
    qt = lax.dot_general(emb, oh, (((0,), (0,)), ((), ())))
    out_ref[0] = qt

    sump_ref[...] += lax.dot_general(ex, r, (((1,), (1,)), ((), ())))
    acc_ref[0] += jnp.sum(ent_row)
    acc_ref[1] += jnp.sum(mind)

    @pl.when(i == _NB - 1)
    def _fin():
        intra = acc_ref[0] / _N
        mse = acc_ref[1] / (_N * _D)
        avg_p = sump_ref[...] / _N
        inter = jnp.sum(avg_p * jnp.log(avg_p + 1e-8))
        lane = lax.broadcasted_iota(jnp.int32, (1, 128), 1)
        stats_ref[...] = jnp.where(
            lane == 0, intra,
            jnp.where(lane == 1, inter, jnp.where(lane == 2, mse, 0.0)))


def kernel(inputs, emb_weight):
    x3 = inputs.reshape(8, _D, _BM)
    q3, stats = pl.pallas_call(
        _vq_body,
        grid=(_NB,),
        in_specs=[
            pl.BlockSpec((1, _D, _BM), lambda i: (i, 0, 0)),
            pl.BlockSpec((_K, _D), lambda i: (0, 0)),
        ],
        out_specs=[
            pl.BlockSpec((1, _D, _BM), lambda i: (i, 0, 0)),
            pl.BlockSpec((1, 128), lambda i: (0, 0)),
        ],
        out_shape=[
            jax.ShapeDtypeStruct((8, _D, _BM), jnp.float32),
            jax.ShapeDtypeStruct((1, 128), jnp.float32),
        ],
        scratch_shapes=[
            pltpu.VMEM((_K, 1), jnp.float32),
            pltpu.SMEM((2,), jnp.float32),
        ],
        compiler_params=pltpu.CompilerParams(
            dimension_semantics=("arbitrary",)),
    )(x3, emb_weight)
    intra = stats[0, 0]
    inter = stats[0, 1]
    mse = stats[0, 2]
    loss = (mse + _E_WEIGHT * mse) + _MANAGE_WEIGHT * (intra + inter)
    out = q3.reshape(8, _D, 32, 32)
    return (loss, out, mse, mse, intra, inter)


# speedup vs baseline: 1.4414x; 1.4414x over previous
import jax
import jax.numpy as jnp
from jax import lax
from jax.experimental import pallas as pl
from jax.experimental.pallas import tpu as pltpu

_K = 1024
_D = 256
_N = 8192
_BM = 1024
_NB = _N // _BM
_E_WEIGHT = 0.25
_MANAGE_WEIGHT = 0.1


def _vq_body(x_ref, emb_ref, q_ref, stats_ref, sump_ref, acc_ref):
    i = pl.program_id(0)

    @pl.when(i == 0)
    def _init():
        sump_ref[...] = jnp.zeros_like(sump_ref)
        acc_ref[0] = 0.0
        acc_ref[1] = 0.0

    x = x_ref[0].reshape(_BM, _D)
    emb = emb_ref[...]
    norm = jnp.sqrt(jnp.sum(x * x, axis=1, keepdims=True))
    xn = x / jnp.maximum(norm, 1e-12)
    s = jnp.sum(xn * xn, axis=1, keepdims=True)
    e2 = jnp.sum(emb * emb, axis=1)
    xm2 = xn * (-2.0)
    dots2 = lax.dot_general(xm2, emb, (((1,), (1,)), ((), ())))
    d = (s + e2[None, :]) + dots2

    ex = jnp.exp(d)
    ones_k = jnp.ones((_K, 1), jnp.float32)
    se = lax.dot_general(ex, ones_k, (((1,), (0,)), ((), ())))
    exd = lax.dot_general(ex * d, ones_k, (((1,), (0,)), ((), ())))
    r = 1.0 / se
    ent_col = jnp.log(se) - exd * r

    mind = jnp.min(d, axis=1, keepdims=True)
    kiof = lax.broadcasted_iota(jnp.int32, d.shape, 1).astype(jnp.float32)
    first = jnp.min(jnp.where(d == mind, kiof, float(_K)),
                    axis=1, keepdims=True)
    oh = (kiof == first).astype(jnp.float32)
    qt = lax.dot_general(oh, emb, (((1,), (0,)), ((), ())))
    q_ref[0] = qt.reshape(32, 32, _D)

    sump_ref[...] += lax.dot_general(r, ex, (((0,), (0,)), ((), ())))
    acc_ref[0] += jnp.sum(ent_col)
    acc_ref[1] += jnp.sum(mind)

    @pl.when(i == _NB - 1)
    def _fin():
        intra = acc_ref[0] / _N
        mse = acc_ref[1] / (_N * _D)
        avg_p = sump_ref[...] / _N
        inter = jnp.sum(avg_p * jnp.log(avg_p + 1e-8))
        lane = lax.broadcasted_iota(jnp.int32, (1, 128), 1)
        stats_ref[...] = jnp.where(
            lane == 0, intra,
            jnp.where(lane == 1, inter, jnp.where(lane == 2, mse, 0.0)))


def kernel(inputs, emb_weight):
    xb = jnp.transpose(inputs, (0, 2, 3, 1))
    q4, stats = pl.pallas_call(
        _vq_body,
        grid=(_NB,),
        in_specs=[
            pl.BlockSpec((1, 32, 32, _D), lambda i: (i, 0, 0, 0)),
            pl.BlockSpec((_K, _D), lambda i: (0, 0)),
        ],
        out_specs=[
            pl.BlockSpec((1, 32, 32, _D), lambda i: (i, 0, 0, 0)),
            pl.BlockSpec((1, 128), lambda i: (0, 0)),
        ],
        out_shape=[
            jax.ShapeDtypeStruct((8, 32, 32, _D), jnp.float32),
            jax.ShapeDtypeStruct((1, 128), jnp.float32),
        ],
        scratch_shapes=[
            pltpu.VMEM((1, _K), jnp.float32),
            pltpu.SMEM((2,), jnp.float32),
        ],
        compiler_params=pltpu.CompilerParams(
            dimension_semantics=("arbitrary",)),
    )(xb, emb_weight)
    intra = stats[0, 0]
    inter = stats[0, 1]
    mse = stats[0, 2]
    loss = (mse + _E_WEIGHT * mse) + _MANAGE_WEIGHT * (intra + inter)
    out = jnp.transpose(q4, (0, 3, 1, 2))
    return (loss, out, mse, mse, intra, inter)


# speedup vs baseline: 1.4610x; 1.0136x over previous
import jax
import jax.numpy as jnp
from jax import lax
from jax.experimental import pallas as pl
from jax.experimental.pallas import tpu as pltpu

_K = 1024
_D = 256
_N = 8192
_BM = 2048
_NB = _N // _BM
_E_WEIGHT = 0.25
_MANAGE_WEIGHT = 0.1


def _vq_body(x_ref, emb_ref, q_ref, stats_ref, sump_ref, acc_ref):
    i = pl.program_id(0)

    @pl.when(i == 0)
    def _init():
        sump_ref[...] = jnp.zeros_like(sump_ref)
        acc_ref[0] = 0.0
        acc_ref[1] = 0.0

    x = x_ref[...].reshape(_BM, _D)
    emb = emb_ref[...]
    norm = jnp.sqrt(jnp.sum(x * x, axis=1, keepdims=True))
    xn = x / jnp.maximum(norm, 1e-12)
    s = jnp.sum(xn * xn, axis=1, keepdims=True)
    e2 = jnp.sum(emb * emb, axis=1)
    xm2 = xn * (-2.0)
    dots2 = lax.dot_general(xm2, emb, (((1,), (1,)), ((), ())))
    d = (s + e2[None, :]) + dots2

    ex = jnp.exp(d)
    ones_k = jnp.ones((_K, 1), jnp.float32)
    se = lax.dot_general(ex, ones_k, (((1,), (0,)), ((), ())))
    exd = lax.dot_general(ex * d, ones_k, (((1,), (0,)), ((), ())))
    r = 1.0 / se
    ent_col = jnp.log(se) - exd * r

    mind = jnp.min(d, axis=1, keepdims=True)
    kiof = lax.broadcasted_iota(jnp.int32, d.shape, 1).astype(jnp.float32)
    first = jnp.min(jnp.where(d == mind, kiof, float(_K)),
                    axis=1, keepdims=True)
    oh = (kiof == first).astype(jnp.float32)
    qt = lax.dot_general(oh, emb, (((1,), (0,)), ((), ())))
    q_ref[...] = qt.reshape(_BM // 1024, 32, 32, _D)

    sump_ref[...] += lax.dot_general(r, ex, (((0,), (0,)), ((), ())))
    acc_ref[0] += jnp.sum(ent_col)
    acc_ref[1] += jnp.sum(mind)

    @pl.when(i == _NB - 1)
    def _fin():
        intra = acc_ref[0] / _N
        mse = acc_ref[1] / (_N * _D)
        avg_p = sump_ref[...] / _N
        inter = jnp.sum(avg_p * jnp.log(avg_p + 1e-8))
        lane = lax.broadcasted_iota(jnp.int32, (1, 128), 1)
        stats_ref[...] = jnp.where(
            lane == 0, intra,
            jnp.where(lane == 1, inter, jnp.where(lane == 2, mse, 0.0)))


def kernel(inputs, emb_weight):
    xb = jnp.transpose(inputs, (0, 2, 3, 1))
    q4, stats = pl.pallas_call(
        _vq_body,
        grid=(_NB,),
        in_specs=[
            pl.BlockSpec((_BM // 1024, 32, 32, _D), lambda i: (i, 0, 0, 0)),
            pl.BlockSpec((_K, _D), lambda i: (0, 0)),
        ],
        out_specs=[
            pl.BlockSpec((_BM // 1024, 32, 32, _D), lambda i: (i, 0, 0, 0)),
            pl.BlockSpec((1, 128), lambda i: (0, 0)),
        ],
        out_shape=[
            jax.ShapeDtypeStruct((8, 32, 32, _D), jnp.float32),
            jax.ShapeDtypeStruct((1, 128), jnp.float32),
        ],
        scratch_shapes=[
            pltpu.VMEM((1, _K), jnp.float32),
            pltpu.SMEM((2,), jnp.float32),
        ],
        compiler_params=pltpu.CompilerParams(
            dimension_semantics=("arbitrary",)),
    )(xb, emb_weight)
    intra = stats[0, 0]
    inter = stats[0, 1]
    mse = stats[0, 2]
    loss = (mse + _E_WEIGHT * mse) + _MANAGE_WEIGHT * (intra + inter)
    out = jnp.transpose(q4, (0, 3, 1, 2))
    return (loss, out, mse, mse, intra, inter)


# speedup vs baseline: 1.5635x; 1.0702x over previous
import jax
import jax.numpy as jnp
from jax import lax
from jax.experimental import pallas as pl
from jax.experimental.pallas import tpu as pltpu

_K = 1024
_D = 256
_N = 8192
_BM = 2048
_NB = _N // _BM
_E_WEIGHT = 0.25
_MANAGE_WEIGHT = 0.1


def _vq_body(x_ref, emb_ref, q_ref, stats_ref, sump_ref, acc_ref):
    i = pl.program_id(0)

    @pl.when(i == 0)
    def _init():
        sump_ref[...] = jnp.zeros_like(sump_ref)
        acc_ref[0] = 0.0
        acc_ref[1] = 0.0

    x = x_ref[...].reshape(_BM, _D)
    emb = emb_ref[...]
    norm = jnp.sqrt(jnp.sum(x * x, axis=1, keepdims=True))
    xn = x / jnp.maximum(norm, 1e-12)
    s = jnp.sum(xn * xn, axis=1, keepdims=True)
    e2 = jnp.sum(emb * emb, axis=1)
    xm2 = xn * (-2.0)
    dots2 = lax.dot_general(xm2, emb, (((1,), (1,)), ((), ())))
    d = (s + e2[None, :]) + dots2

    ex = jnp.exp(d)
    se = jnp.sum(ex, axis=1, keepdims=True)
    exd = jnp.sum(ex * d, axis=1, keepdims=True)
    r = 1.0 / se
    ent_col = jnp.log(se) - exd * r

    mind = jnp.min(d, axis=1, keepdims=True)
    kiof = lax.broadcasted_iota(jnp.int32, (1, _K), 1).astype(jnp.float32)
    first = jnp.min(jnp.where(d == mind, kiof, float(_K)),
                    axis=1, keepdims=True)
    oh = (kiof == first).astype(jnp.float32)
    qt = lax.dot_general(oh, emb, (((1,), (0,)), ((), ())))
    q_ref[...] = qt.reshape(_BM // 1024, 32, 32, _D)

    sump_ref[...] += lax.dot_general(r, ex, (((0,), (0,)), ((), ())))
    acc_ref[0] += jnp.sum(ent_col)
    acc_ref[1] += jnp.sum(mind)

    @pl.when(i == _NB - 1)
    def _fin():
        intra = acc_ref[0] / _N
        mse = acc_ref[1] / (_N * _D)
        avg_p = sump_ref[...] / _N
        inter = jnp.sum(avg_p * jnp.log(avg_p + 1e-8))
        lane = lax.broadcasted_iota(jnp.int32, (1, 128), 1)
        stats_ref[...] = jnp.where(
            lane == 0, intra,
            jnp.where(lane == 1, inter, jnp.where(lane == 2, mse, 0.0)))


def kernel(inputs, emb_weight):
    xb = jnp.transpose(inputs, (0, 2, 3, 1))
    q4, stats = pl.pallas_call(
        _vq_body,
        grid=(_NB,),
        in_specs=[
            pl.BlockSpec((_BM // 1024, 32, 32, _D), lambda i: (i, 0, 0, 0)),
            pl.BlockSpec((_K, _D), lambda i: (0, 0)),
        ],
        out_specs=[
            pl.BlockSpec((_BM // 1024, 32, 32, _D), lambda i: (i, 0, 0, 0)),
            pl.BlockSpec((1, 128), lambda i: (0, 0)),
        ],
        out_shape=[
            jax.ShapeDtypeStruct((8, 32, 32, _D), jnp.float32),
            jax.ShapeDtypeStruct((1, 128), jnp.float32),
        ],
        scratch_shapes=[
            pltpu.VMEM((1, _K), jnp.float32),
            pltpu.SMEM((2,), jnp.float32),
        ],
        compiler_params=pltpu.CompilerParams(
            dimension_semantics=("arbitrary",)),
    )(xb, emb_weight)
    intra = stats[0, 0]
    inter = stats[0, 1]
    mse = stats[0, 2]
    loss = (mse + _E_WEIGHT * mse) + _MANAGE_WEIGHT * (intra + inter)
    out = jnp.transpose(q4, (0, 3, 1, 2))
    return (loss, out, mse, mse, intra, inter)
